# baseline (speedup 1.0000x reference)
import jax
import jax.numpy as jnp
from jax import lax
from jax.experimental import pallas as pl
from jax.experimental.pallas import tpu as pltpu
from jax.experimental.pallas import tpu_sc as plsc

N = 10000
E = 320000
D = 128
C = 7

NC = 2
NS = 16
NW = NC * NS
NP = 10240
EPW = 10240
K = 128
NCH = EPW // K
IB = 16
NIB = NCH // IB
RPT = NP // NS
RCH = 80
NRCH = RPT // RCH
HL = EPW // 16

_mesh = plsc.VectorSubcoreMesh(
    core_axis_name="c", subcore_axis_name="s", num_cores=NC, num_subcores=NS)


def _sc_deg_body(dstf_hbm, zflat_hbm, degp_hbm, dstf_v, hist_v):
    c = lax.axis_index("c")
    s = lax.axis_index("s")
    w = c * NS + s

    pltpu.sync_copy(dstf_hbm.at[w], dstf_v)
    pltpu.sync_copy(zflat_hbm, hist_v)
    ones16 = jnp.ones((16,), jnp.float32)

    def hstep(i, carry):
        idx = dstf_v[pl.ds(pl.multiple_of(i * 16, 16), 16)]
        plsc.addupdate_scatter(hist_v, [idx], ones16)
        return carry

    lax.fori_loop(0, HL, hstep, 0)
    pltpu.sync_copy(hist_v, degp_hbm.at[w])


_sc_deg = pl.kernel(
    _sc_deg_body,
    out_type=jax.ShapeDtypeStruct((NW, NP), jnp.float32),
    mesh=_mesh,
    scratch_types=[
        pltpu.VMEM((EPW,), jnp.int32),
        pltpu.VMEM((NP,), jnp.float32),
    ],
    compiler_params=pltpu.CompilerParams(needs_layout_passes=False),
)


def _sc_agg_body(h_hbm, src_hbm, dst_hbm, zrow_hbm,
                 part_hbm,
                 src_v, dst_v, rows_v, acc, gsem, ssem):
    c = lax.axis_index("c")
    s = lax.axis_index("s")
    w = c * NS + s

    pltpu.sync_copy(zrow_hbm, rows_v.at[0, pl.ds(0, RCH)])
    for r in range(NRCH):
        base = s * RPT + r * RCH
        pltpu.sync_copy(rows_v.at[0, pl.ds(0, RCH)],
                        acc.at[pl.ds(base, RCH)])
    plsc.subcore_barrier()

    def start_gather(b, rslot):
        pltpu.async_copy(h_hbm.at[src_v.at[b]], rows_v.at[rslot], gsem)

    def wait_gather(b, rslot):
        pltpu.make_async_copy(h_hbm.at[src_v.at[b]], rows_v.at[rslot],
                              gsem).wait()

    def start_scatter(b, rslot):
        pltpu.async_copy(rows_v.at[rslot], acc.at[dst_v.at[b]], ssem,
                         add=True)

    def wait_scatter(b, rslot):
        pltpu.make_async_copy(rows_v.at[rslot], acc.at[dst_v.at[b]],
                              ssem).wait()

    def group(g, carry):
        gb = pl.multiple_of(g * IB, IB)

        @pl.when(g > 0)
        def _():
            wait_scatter(IB - 1, (IB - 1) % 2)

        pltpu.sync_copy(src_hbm.at[w, pl.ds(gb, IB)], src_v)
        pltpu.sync_copy(dst_hbm.at[w, pl.ds(gb, IB)], dst_v)
        start_gather(0, 0)
        for b in range(IB):
            wait_gather(b, b % 2)
            if b > 0:
                wait_scatter(b - 1, (b - 1) % 2)
            start_scatter(b, b % 2)
            if b + 1 < IB:
                start_gather(b + 1, (b + 1) % 2)
        return carry

    lax.fori_loop(0, NIB, group, 0)
    wait_scatter(IB - 1, (IB - 1) % 2)
    plsc.subcore_barrier()

    for r in range(NRCH):
        base = s * RPT + r * RCH
        pltpu.sync_copy(acc.at[pl.ds(base, RCH)], rows_v.at[0, pl.ds(0, RCH)])
        pltpu.sync_copy(rows_v.at[0, pl.ds(0, RCH)],
                        part_hbm.at[c, pl.ds(base, RCH)])


_sc_agg = pl.kernel(
    _sc_agg_body,
    out_type=jax.ShapeDtypeStruct((NC, NP, D), jnp.float32),
    mesh=_mesh,
    scratch_types=[
        pltpu.VMEM((IB, K), jnp.int32),
        pltpu.VMEM((IB, K), jnp.int32),
        pltpu.VMEM((2, K, D), jnp.float32),
        pltpu.VMEM_SHARED((NP, D), jnp.float32),
        pltpu.SemaphoreType.DMA,
        pltpu.SemaphoreType.DMA,
    ],
    compiler_params=pltpu.CompilerParams(needs_layout_passes=False),
)


BR = 1000
GRID = N // BR
BP = NP // GRID


def _tc_lin1(x_ref, w_ref, b_ref, o_ref):
    o_ref[...] = (
        jnp.dot(x_ref[...], w_ref[...], preferred_element_type=jnp.float32)
        + b_ref[...])


def _tc_lin2(p_ref, dg_ref, w_ref, b_ref, o_ref):
    agg = p_ref[0] + p_ref[1]
    deg = jnp.maximum(jnp.sum(dg_ref[...], axis=0), 1.0)[:, None]
    h = jnp.maximum(agg / deg, 0.0)
    o_ref[...] = (
        jnp.dot(h, w_ref[...], preferred_element_type=jnp.float32)
        + b_ref[...])


def _tc_decode(p_ref, dg_ref, w_ref, b_ref, o_ref):
    agg = p_ref[0] + p_ref[1]
    deg = jnp.maximum(jnp.sum(dg_ref[...], axis=0), 1.0)[:, None]
    h = agg / deg
    logits = (
        jnp.dot(h, w_ref[...], preferred_element_type=jnp.float32)
        + b_ref[...])
    m = jnp.max(logits, axis=-1, keepdims=True)
    ex = jnp.exp(logits - m)
    lse = jnp.log(jnp.sum(ex, axis=-1, keepdims=True))
    o_ref[...] = logits - m - lse


def kernel(x, edge_index, W1, b1, W2, b2, Wd, bd):
    ppw = EPW - E // NW
    srcp = jnp.concatenate(
        [edge_index[0].reshape(NW, E // NW),
         jnp.zeros((NW, ppw), jnp.int32)], axis=1)
    pad_dst = jnp.broadcast_to(N + jnp.arange(ppw, dtype=jnp.int32),
                               (NW, ppw))
    dstp = jnp.concatenate(
        [edge_index[1].reshape(NW, E // NW), pad_dst], axis=1)
    src = srcp.reshape(NW, NCH, K)
    dst3 = dstp.reshape(NW, NCH, K)
    dstf = dstp.reshape(NW, EPW)
    zrow = jnp.zeros((RCH, D), jnp.float32)
    zflat = jnp.zeros((NP,), jnp.float32)

    degp = _sc_deg(dstf, zflat)

    h1 = pl.pallas_call(
        _tc_lin1,
        grid=(GRID,),
        in_specs=[
            pl.BlockSpec((BR, D), lambda i: (i, 0)),
            pl.BlockSpec((D, D), lambda i: (0, 0)),
            pl.BlockSpec((1, D), lambda i: (0, 0)),
        ],
        out_specs=pl.BlockSpec((BR, D), lambda i: (i, 0)),
        out_shape=jax.ShapeDtypeStruct((N, D), jnp.float32),
    )(x, W1, b1.reshape(1, D))

    part1 = _sc_agg(h1, src, dst3, zrow)

    h2 = pl.pallas_call(
        _tc_lin2,
        grid=(GRID,),
        in_specs=[
            pl.BlockSpec((NC, BP, D), lambda i: (0, i, 0)),
            pl.BlockSpec((NW, BP), lambda i: (0, i)),
            pl.BlockSpec((D, D), lambda i: (0, 0)),
            pl.BlockSpec((1, D), lambda i: (0, 0)),
        ],
        out_specs=pl.BlockSpec((BP, D), lambda i: (i, 0)),
        out_shape=jax.ShapeDtypeStruct((NP, D), jnp.float32),
    )(part1, degp, W2, b2.reshape(1, D))

    part2 = _sc_agg(h2, src, dst3, zrow)

    out = pl.pallas_call(
        _tc_decode,
        grid=(GRID,),
        in_specs=[
            pl.BlockSpec((NC, BP, D), lambda i: (0, i, 0)),
            pl.BlockSpec((NW, BP), lambda i: (0, i)),
            pl.BlockSpec((D, C), lambda i: (0, 0)),
            pl.BlockSpec((1, C), lambda i: (0, 0)),
        ],
        out_specs=pl.BlockSpec((BP, C), lambda i: (i, 0)),
        out_shape=jax.ShapeDtypeStruct((NP, C), jnp.float32),
    )(part2, degp, Wd, bd.reshape(1, C))

    return out[:N]


# speedup vs baseline: 2.5603x; 2.5603x over previous
import jax
import jax.numpy as jnp
from jax import lax
from jax.experimental import pallas as pl
from jax.experimental.pallas import tpu as pltpu
from jax.experimental.pallas import tpu_sc as plsc

N = 10000
E = 320000
D = 128
C = 7

NC = 2
NS = 16
NW = NC * NS
NP = 10240
EPW = E // NW
K = 125
NCH = EPW // K
IB = 16
NIB = NCH // IB
RPT = NP // NS
RCH = 80
NRCH = RPT // RCH
HL = EPW // 16

_mesh = plsc.VectorSubcoreMesh(
    core_axis_name="c", subcore_axis_name="s", num_cores=NC, num_subcores=NS)


def _sc_deg_body(dstf_hbm, zflat_hbm, degp_hbm, dstf_v, hist_v):
    c = lax.axis_index("c")
    s = lax.axis_index("s")
    w = c * NS + s

    pltpu.sync_copy(dstf_hbm.at[w], dstf_v)
    pltpu.sync_copy(zflat_hbm, hist_v)
    ones16 = jnp.ones((16,), jnp.float32)

    def hstep(i, carry):
        idx = dstf_v[pl.ds(pl.multiple_of(i * 16, 16), 16)]
        plsc.addupdate_scatter(hist_v, [idx], ones16)
        return carry

    lax.fori_loop(0, HL, hstep, 0)
    pltpu.sync_copy(hist_v, degp_hbm.at[w])


_sc_deg = pl.kernel(
    _sc_deg_body,
    out_type=jax.ShapeDtypeStruct((NW, NP), jnp.float32),
    mesh=_mesh,
    scratch_types=[
        pltpu.VMEM((EPW,), jnp.int32),
        pltpu.VMEM((NP,), jnp.float32),
    ],
    compiler_params=pltpu.CompilerParams(needs_layout_passes=False),
)


def _sc_agg_body(h_hbm, src_hbm, dst_hbm, zrow_hbm,
                 part_hbm,
                 src_v, dst_v, rows_v, acc, gsem, ssem):
    c = lax.axis_index("c")
    s = lax.axis_index("s")
    w = c * NS + s

    pltpu.sync_copy(zrow_hbm, rows_v.at[0, pl.ds(0, RCH)])
    for r in range(NRCH):
        base = s * RPT + r * RCH
        pltpu.sync_copy(rows_v.at[0, pl.ds(0, RCH)],
                        acc.at[pl.ds(base, RCH)])
    plsc.subcore_barrier()

    def start_gather(b, rslot):
        pltpu.async_copy(h_hbm.at[src_v.at[b]], rows_v.at[rslot], gsem)

    def wait_gather(b, rslot):
        pltpu.make_async_copy(h_hbm.at[src_v.at[b]], rows_v.at[rslot],
                              gsem).wait()

    def start_scatter(b, rslot):
        pltpu.async_copy(rows_v.at[rslot], acc.at[dst_v.at[b]], ssem,
                         add=True)

    def wait_scatter(b, rslot):
        pltpu.make_async_copy(rows_v.at[rslot], acc.at[dst_v.at[b]],
                              ssem).wait()

    def group(g, carry):
        gb = pl.multiple_of(g * IB, IB)

        @pl.when(g > 0)
        def _():
            wait_scatter(IB - 1, (IB - 1) % 2)

        pltpu.sync_copy(src_hbm.at[w, pl.ds(gb, IB)], src_v)
        pltpu.sync_copy(dst_hbm.at[w, pl.ds(gb, IB)], dst_v)
        start_gather(0, 0)
        for b in range(IB):
            wait_gather(b, b % 2)
            if b > 0:
                wait_scatter(b - 1, (b - 1) % 2)
            start_scatter(b, b % 2)
            if b + 1 < IB:
                start_gather(b + 1, (b + 1) % 2)
        return carry

    lax.fori_loop(0, NIB, group, 0)
    wait_scatter(IB - 1, (IB - 1) % 2)
    plsc.subcore_barrier()

    for r in range(NRCH):
        base = s * RPT + r * RCH
        pltpu.sync_copy(acc.at[pl.ds(base, RCH)], rows_v.at[0, pl.ds(0, RCH)])
        pltpu.sync_copy(rows_v.at[0, pl.ds(0, RCH)],
                        part_hbm.at[c, pl.ds(base, RCH)])


_sc_agg = pl.kernel(
    _sc_agg_body,
    out_type=jax.ShapeDtypeStruct((NC, NP, D), jnp.float32),
    mesh=_mesh,
    scratch_types=[
        pltpu.VMEM((IB, K), jnp.int32),
        pltpu.VMEM((IB, K), jnp.int32),
        pltpu.VMEM((2, K, D), jnp.float32),
        pltpu.VMEM_SHARED((NP, D), jnp.float32),
        pltpu.SemaphoreType.DMA,
        pltpu.SemaphoreType.DMA,
    ],
    compiler_params=pltpu.CompilerParams(needs_layout_passes=False),
)


BR = 1000
GRID = N // BR
BP = NP // GRID


def _tc_lin1(x_ref, w_ref, b_ref, o_ref):
    o_ref[...] = (
        jnp.dot(x_ref[...], w_ref[...], preferred_element_type=jnp.float32)
        + b_ref[...])


def _tc_lin2(p_ref, dg_ref, w_ref, b_ref, o_ref):
    agg = p_ref[0] + p_ref[1]
    deg = jnp.maximum(jnp.sum(dg_ref[...], axis=0), 1.0)[:, None]
    h = jnp.maximum(agg / deg, 0.0)
    o_ref[...] = (
        jnp.dot(h, w_ref[...], preferred_element_type=jnp.float32)
        + b_ref[...])


def _tc_decode(p_ref, dg_ref, w_ref, b_ref, o_ref):
    agg = p_ref[0] + p_ref[1]
    deg = jnp.maximum(jnp.sum(dg_ref[...], axis=0), 1.0)[:, None]
    h = agg / deg
    logits = (
        jnp.dot(h, w_ref[...], preferred_element_type=jnp.float32)
        + b_ref[...])
    m = jnp.max(logits, axis=-1, keepdims=True)
    ex = jnp.exp(logits - m)
    lse = jnp.log(jnp.sum(ex, axis=-1, keepdims=True))
    o_ref[...] = logits - m - lse


def kernel(x, edge_index, W1, b1, W2, b2, Wd, bd):
    src = edge_index[0].reshape(NW, NCH, K)
    dst3 = edge_index[1].reshape(NW, NCH, K)
    dstf = edge_index[1].reshape(NW, EPW)
    zrow = jnp.zeros((RCH, D), jnp.float32)
    zflat = jnp.zeros((NP,), jnp.float32)

    degp = _sc_deg(dstf, zflat)

    h1 = pl.pallas_call(
        _tc_lin1,
        grid=(GRID,),
        in_specs=[
            pl.BlockSpec((BR, D), lambda i: (i, 0)),
            pl.BlockSpec((D, D), lambda i: (0, 0)),
            pl.BlockSpec((1, D), lambda i: (0, 0)),
        ],
        out_specs=pl.BlockSpec((BR, D), lambda i: (i, 0)),
        out_shape=jax.ShapeDtypeStruct((N, D), jnp.float32),
    )(x, W1, b1.reshape(1, D))

    part1 = _sc_agg(h1, src, dst3, zrow)

    h2 = pl.pallas_call(
        _tc_lin2,
        grid=(GRID,),
        in_specs=[
            pl.BlockSpec((NC, BP, D), lambda i: (0, i, 0)),
            pl.BlockSpec((NW, BP), lambda i: (0, i)),
            pl.BlockSpec((D, D), lambda i: (0, 0)),
            pl.BlockSpec((1, D), lambda i: (0, 0)),
        ],
        out_specs=pl.BlockSpec((BP, D), lambda i: (i, 0)),
        out_shape=jax.ShapeDtypeStruct((NP, D), jnp.float32),
    )(part1, degp, W2, b2.reshape(1, D))

    part2 = _sc_agg(h2, src, dst3, zrow)

    out = pl.pallas_call(
        _tc_decode,
        grid=(GRID,),
        in_specs=[
            pl.BlockSpec((NC, BP, D), lambda i: (0, i, 0)),
            pl.BlockSpec((NW, BP), lambda i: (0, i)),
            pl.BlockSpec((D, C), lambda i: (0, 0)),
            pl.BlockSpec((1, C), lambda i: (0, 0)),
        ],
        out_specs=pl.BlockSpec((BP, C), lambda i: (i, 0)),
        out_shape=jax.ShapeDtypeStruct((NP, C), jnp.float32),
    )(part2, degp, Wd, bd.reshape(1, C))

    return out[:N]


# speedup vs baseline: 2.5663x; 1.0024x over previous
import jax
import jax.numpy as jnp
from jax import lax
from jax.experimental import pallas as pl
from jax.experimental.pallas import tpu as pltpu
from jax.experimental.pallas import tpu_sc as plsc

N = 10000
E = 320000
D = 128
C = 7

NC = 2
NS = 16
NW = NC * NS
NP = 10240
EPW = E // NW
K = 125
NCH = EPW // K
IB = 16
NIB = NCH // IB
RPT = NP // NS
RCH = 80
NRCH = RPT // RCH
HL = EPW // 16

_mesh = plsc.VectorSubcoreMesh(
    core_axis_name="c", subcore_axis_name="s", num_cores=NC, num_subcores=NS)


def _sc_deg_body(dstf_hbm, zflat_hbm, degp_hbm, dstf_v, hist_v):
    c = lax.axis_index("c")
    s = lax.axis_index("s")
    w = c * NS + s

    pltpu.sync_copy(dstf_hbm.at[w], dstf_v)
    pltpu.sync_copy(zflat_hbm, hist_v)
    ones16 = jnp.ones((16,), jnp.float32)

    def hstep(i, carry):
        idx = dstf_v[pl.ds(pl.multiple_of(i * 16, 16), 16)]
        plsc.addupdate_scatter(hist_v, [idx], ones16)
        return carry

    lax.fori_loop(0, HL, hstep, 0)
    pltpu.sync_copy(hist_v, degp_hbm.at[w])


_sc_deg = pl.kernel(
    _sc_deg_body,
    out_type=jax.ShapeDtypeStruct((NW, NP), jnp.float32),
    mesh=_mesh,
    scratch_types=[
        pltpu.VMEM((EPW,), jnp.int32),
        pltpu.VMEM((NP,), jnp.float32),
    ],
    compiler_params=pltpu.CompilerParams(needs_layout_passes=False),
)


def _sc_agg_body(h_hbm, src_hbm, dst_hbm, zrow_hbm,
                 part_hbm,
                 src_v, dst_v, rows_v, acc, gsem, ssem):
    c = lax.axis_index("c")
    s = lax.axis_index("s")
    w = c * NS + s

    pltpu.sync_copy(zrow_hbm, rows_v.at[0, pl.ds(0, RCH)])
    for r in range(NRCH):
        base = s * RPT + r * RCH
        pltpu.sync_copy(rows_v.at[0, pl.ds(0, RCH)],
                        acc.at[pl.ds(base, RCH)])
    plsc.subcore_barrier()

    def start_gather(b, rslot):
        pltpu.async_copy(h_hbm.at[src_v.at[b]], rows_v.at[rslot], gsem)

    def wait_gather(b, rslot):
        pltpu.make_async_copy(h_hbm.at[src_v.at[b]], rows_v.at[rslot],
                              gsem).wait()

    def start_scatter(b, rslot):
        pltpu.async_copy(rows_v.at[rslot], acc.at[dst_v.at[b]], ssem,
                         add=True)

    def wait_scatter(b, rslot):
        pltpu.make_async_copy(rows_v.at[rslot], acc.at[dst_v.at[b]],
                              ssem).wait()

    def group(g, carry):
        gb = pl.multiple_of(g * IB, IB)

        @pl.when(g > 0)
        def _():
            wait_scatter(IB - 1, (IB - 1) % 2)

        pltpu.sync_copy(src_hbm.at[w, pl.ds(gb, IB)], src_v)
        pltpu.sync_copy(dst_hbm.at[w, pl.ds(gb, IB)], dst_v)
        start_gather(0, 0)
        for b in range(IB):
            if b > 0:
                wait_scatter(b - 1, (b - 1) % 2)
            wait_gather(b, b % 2)
            if b + 1 < IB:
                start_gather(b + 1, (b + 1) % 2)
            start_scatter(b, b % 2)
        return carry

    lax.fori_loop(0, NIB, group, 0)
    wait_scatter(IB - 1, (IB - 1) % 2)
    plsc.subcore_barrier()

    for r in range(NRCH):
        base = s * RPT + r * RCH
        pltpu.sync_copy(acc.at[pl.ds(base, RCH)], rows_v.at[0, pl.ds(0, RCH)])
        pltpu.sync_copy(rows_v.at[0, pl.ds(0, RCH)],
                        part_hbm.at[c, pl.ds(base, RCH)])


_sc_agg = pl.kernel(
    _sc_agg_body,
    out_type=jax.ShapeDtypeStruct((NC, NP, D), jnp.float32),
    mesh=_mesh,
    scratch_types=[
        pltpu.VMEM((IB, K), jnp.int32),
        pltpu.VMEM((IB, K), jnp.int32),
        pltpu.VMEM((2, K, D), jnp.float32),
        pltpu.VMEM_SHARED((NP, D), jnp.float32),
        pltpu.SemaphoreType.DMA,
        pltpu.SemaphoreType.DMA,
    ],
    compiler_params=pltpu.CompilerParams(needs_layout_passes=False),
)


BR = 1000
GRID = N // BR
BP = NP // GRID


def _tc_lin1(x_ref, w_ref, b_ref, o_ref):
    o_ref[...] = (
        jnp.dot(x_ref[...], w_ref[...], preferred_element_type=jnp.float32)
        + b_ref[...])


def _tc_lin2(p_ref, dg_ref, w_ref, b_ref, o_ref):
    agg = p_ref[0] + p_ref[1]
    deg = jnp.maximum(jnp.sum(dg_ref[...], axis=0), 1.0)[:, None]
    h = jnp.maximum(agg / deg, 0.0)
    o_ref[...] = (
        jnp.dot(h, w_ref[...], preferred_element_type=jnp.float32)
        + b_ref[...])


def _tc_decode(p_ref, dg_ref, w_ref, b_ref, o_ref):
    agg = p_ref[0] + p_ref[1]
    deg = jnp.maximum(jnp.sum(dg_ref[...], axis=0), 1.0)[:, None]
    h = agg / deg
    logits = (
        jnp.dot(h, w_ref[...], preferred_element_type=jnp.float32)
        + b_ref[...])
    m = jnp.max(logits, axis=-1, keepdims=True)
    ex = jnp.exp(logits - m)
    lse = jnp.log(jnp.sum(ex, axis=-1, keepdims=True))
    o_ref[...] = logits - m - lse


def kernel(x, edge_index, W1, b1, W2, b2, Wd, bd):
    src = edge_index[0].reshape(NW, NCH, K)
    dst3 = edge_index[1].reshape(NW, NCH, K)
    dstf = edge_index[1].reshape(NW, EPW)
    zrow = jnp.zeros((RCH, D), jnp.float32)
    zflat = jnp.zeros((NP,), jnp.float32)

    degp = _sc_deg(dstf, zflat)

    h1 = pl.pallas_call(
        _tc_lin1,
        grid=(GRID,),
        in_specs=[
            pl.BlockSpec((BR, D), lambda i: (i, 0)),
            pl.BlockSpec((D, D), lambda i: (0, 0)),
            pl.BlockSpec((1, D), lambda i: (0, 0)),
        ],
        out_specs=pl.BlockSpec((BR, D), lambda i: (i, 0)),
        out_shape=jax.ShapeDtypeStruct((N, D), jnp.float32),
    )(x, W1, b1.reshape(1, D))

    part1 = _sc_agg(h1, src, dst3, zrow)

    h2 = pl.pallas_call(
        _tc_lin2,
        grid=(GRID,),
        in_specs=[
            pl.BlockSpec((NC, BP, D), lambda i: (0, i, 0)),
            pl.BlockSpec((NW, BP), lambda i: (0, i)),
            pl.BlockSpec((D, D), lambda i: (0, 0)),
            pl.BlockSpec((1, D), lambda i: (0, 0)),
        ],
        out_specs=pl.BlockSpec((BP, D), lambda i: (i, 0)),
        out_shape=jax.ShapeDtypeStruct((NP, D), jnp.float32),
    )(part1, degp, W2, b2.reshape(1, D))

    part2 = _sc_agg(h2, src, dst3, zrow)

    out = pl.pallas_call(
        _tc_decode,
        grid=(GRID,),
        in_specs=[
            pl.BlockSpec((NC, BP, D), lambda i: (0, i, 0)),
            pl.BlockSpec((NW, BP), lambda i: (0, i)),
            pl.BlockSpec((D, C), lambda i: (0, 0)),
            pl.BlockSpec((1, C), lambda i: (0, 0)),
        ],
        out_specs=pl.BlockSpec((BP, C), lambda i: (i, 0)),
        out_shape=jax.ShapeDtypeStruct((NP, C), jnp.float32),
    )(part2, degp, Wd, bd.reshape(1, C))

    return out[:N]


# speedup vs baseline: 2.6128x; 1.0181x over previous
import jax
import jax.numpy as jnp
from jax import lax
from jax.experimental import pallas as pl
from jax.experimental.pallas import tpu as pltpu
from jax.experimental.pallas import tpu_sc as plsc

N = 10000
E = 320000
D = 128
C = 7

NC = 2
NS = 16
NW = NC * NS
NP = 10240
EPW = E // NW
K = 125
NCH = EPW // K
IB = 16
NIB = NCH // IB
RPT = NP // NS
RCH = 80
NRCH = RPT // RCH
HL = EPW // 16

_mesh = plsc.VectorSubcoreMesh(
    core_axis_name="c", subcore_axis_name="s", num_cores=NC, num_subcores=NS)


def _sc_deg_body(dstf_hbm, zflat_hbm, degp_hbm, dstf_v, hist_v):
    c = lax.axis_index("c")
    s = lax.axis_index("s")
    w = c * NS + s

    pltpu.sync_copy(dstf_hbm.at[w], dstf_v)
    pltpu.sync_copy(zflat_hbm, hist_v)
    ones16 = jnp.ones((16,), jnp.float32)

    def hstep(i, carry):
        idx = dstf_v[pl.ds(pl.multiple_of(i * 16, 16), 16)]
        plsc.addupdate_scatter(hist_v, [idx], ones16)
        return carry

    lax.fori_loop(0, HL, hstep, 0)
    pltpu.sync_copy(hist_v, degp_hbm.at[w])


_sc_deg = pl.kernel(
    _sc_deg_body,
    out_type=jax.ShapeDtypeStruct((NW, NP), jnp.float32),
    mesh=_mesh,
    scratch_types=[
        pltpu.VMEM((EPW,), jnp.int32),
        pltpu.VMEM((NP,), jnp.float32),
    ],
    compiler_params=pltpu.CompilerParams(needs_layout_passes=False),
)


def _sc_agg_body(h_hbm, src_hbm, dst_hbm, zrow_hbm,
                 part_hbm,
                 src_v, dst_v, rows_v, acc, gsem, ssem):
    c = lax.axis_index("c")
    s = lax.axis_index("s")
    w = c * NS + s

    def start_gather(b, rslot):
        pltpu.async_copy(h_hbm.at[src_v.at[b]], rows_v.at[rslot], gsem)

    def wait_gather(b, rslot):
        pltpu.make_async_copy(h_hbm.at[src_v.at[b]], rows_v.at[rslot],
                              gsem).wait()

    def start_scatter(b, rslot):
        pltpu.async_copy(rows_v.at[rslot], acc.at[dst_v.at[b]], ssem,
                         add=True)

    def wait_scatter(b, rslot):
        pltpu.make_async_copy(rows_v.at[rslot], acc.at[dst_v.at[b]],
                              ssem).wait()

    pltpu.sync_copy(src_hbm.at[w, pl.ds(0, IB)], src_v)
    pltpu.sync_copy(dst_hbm.at[w, pl.ds(0, IB)], dst_v)
    start_gather(0, 0)
    pltpu.sync_copy(zrow_hbm, rows_v.at[1, pl.ds(0, RCH)])
    for r in range(NRCH):
        base = s * RPT + r * RCH
        pltpu.sync_copy(rows_v.at[1, pl.ds(0, RCH)],
                        acc.at[pl.ds(base, RCH)])
    plsc.subcore_barrier()

    def group(g, carry):
        gb = pl.multiple_of(g * IB, IB)

        @pl.when(g > 0)
        def _():
            wait_scatter(IB - 1, (IB - 1) % 2)
            pltpu.sync_copy(src_hbm.at[w, pl.ds(gb, IB)], src_v)
            pltpu.sync_copy(dst_hbm.at[w, pl.ds(gb, IB)], dst_v)
            start_gather(0, 0)

        for b in range(IB):
            if b > 0:
                wait_scatter(b - 1, (b - 1) % 2)
            wait_gather(b, b % 2)
            if b + 1 < IB:
                start_gather(b + 1, (b + 1) % 2)
            start_scatter(b, b % 2)
        return carry

    lax.fori_loop(0, NIB, group, 0)
    wait_scatter(IB - 1, (IB - 1) % 2)
    plsc.subcore_barrier()

    def rd(r, rslot):
        base = s * RPT + r * RCH
        return pltpu.make_async_copy(acc.at[pl.ds(base, RCH)],
                                     rows_v.at[rslot, pl.ds(0, RCH)], gsem)

    pltpu.sync_copy(acc.at[pl.ds(s * RPT, RCH)], rows_v.at[0, pl.ds(0, RCH)])
    for r in range(NRCH):
        if r + 1 < NRCH:
            pltpu.async_copy(acc.at[pl.ds(s * RPT + (r + 1) * RCH, RCH)],
                             rows_v.at[(r + 1) % 2, pl.ds(0, RCH)], gsem)
        base = s * RPT + r * RCH
        pltpu.sync_copy(rows_v.at[r % 2, pl.ds(0, RCH)],
                        part_hbm.at[c, pl.ds(base, RCH)])
        if r + 1 < NRCH:
            rd(r + 1, (r + 1) % 2).wait()


_sc_agg = pl.kernel(
    _sc_agg_body,
    out_type=jax.ShapeDtypeStruct((NC, NP, D), jnp.float32),
    mesh=_mesh,
    scratch_types=[
        pltpu.VMEM((IB, K), jnp.int32),
        pltpu.VMEM((IB, K), jnp.int32),
        pltpu.VMEM((2, K, D), jnp.float32),
        pltpu.VMEM_SHARED((NP, D), jnp.float32),
        pltpu.SemaphoreType.DMA,
        pltpu.SemaphoreType.DMA,
    ],
    compiler_params=pltpu.CompilerParams(needs_layout_passes=False),
)


BR = 1000
GRID = N // BR
BP = NP // GRID


def _tc_lin1(x_ref, w_ref, b_ref, o_ref):
    o_ref[...] = (
        jnp.dot(x_ref[...], w_ref[...], preferred_element_type=jnp.float32)
        + b_ref[...])


def _tc_lin2(p_ref, dg_ref, w_ref, b_ref, o_ref):
    agg = p_ref[0] + p_ref[1]
    deg = jnp.maximum(jnp.sum(dg_ref[...], axis=0), 1.0)[:, None]
    h = jnp.maximum(agg / deg, 0.0)
    o_ref[...] = (
        jnp.dot(h, w_ref[...], preferred_element_type=jnp.float32)
        + b_ref[...])


def _tc_decode(p_ref, dg_ref, w_ref, b_ref, o_ref):
    agg = p_ref[0] + p_ref[1]
    deg = jnp.maximum(jnp.sum(dg_ref[...], axis=0), 1.0)[:, None]
    h = agg / deg
    logits = (
        jnp.dot(h, w_ref[...], preferred_element_type=jnp.float32)
        + b_ref[...])
    m = jnp.max(logits, axis=-1, keepdims=True)
    ex = jnp.exp(logits - m)
    lse = jnp.log(jnp.sum(ex, axis=-1, keepdims=True))
    o_ref[...] = logits - m - lse


def kernel(x, edge_index, W1, b1, W2, b2, Wd, bd):
    src = edge_index[0].reshape(NW, NCH, K)
    dst3 = edge_index[1].reshape(NW, NCH, K)
    dstf = edge_index[1].reshape(NW, EPW)
    zrow = jnp.zeros((RCH, D), jnp.float32)
    zflat = jnp.zeros((NP,), jnp.float32)

    degp = _sc_deg(dstf, zflat)

    h1 = pl.pallas_call(
        _tc_lin1,
        grid=(GRID,),
        in_specs=[
            pl.BlockSpec((BR, D), lambda i: (i, 0)),
            pl.BlockSpec((D, D), lambda i: (0, 0)),
            pl.BlockSpec((1, D), lambda i: (0, 0)),
        ],
        out_specs=pl.BlockSpec((BR, D), lambda i: (i, 0)),
        out_shape=jax.ShapeDtypeStruct((N, D), jnp.float32),
    )(x, W1, b1.reshape(1, D))

    part1 = _sc_agg(h1, src, dst3, zrow)

    h2 = pl.pallas_call(
        _tc_lin2,
        grid=(GRID,),
        in_specs=[
            pl.BlockSpec((NC, BP, D), lambda i: (0, i, 0)),
            pl.BlockSpec((NW, BP), lambda i: (0, i)),
            pl.BlockSpec((D, D), lambda i: (0, 0)),
            pl.BlockSpec((1, D), lambda i: (0, 0)),
        ],
        out_specs=pl.BlockSpec((BP, D), lambda i: (i, 0)),
        out_shape=jax.ShapeDtypeStruct((NP, D), jnp.float32),
    )(part1, degp, W2, b2.reshape(1, D))

    part2 = _sc_agg(h2, src, dst3, zrow)

    out = pl.pallas_call(
        _tc_decode,
        grid=(GRID,),
        in_specs=[
            pl.BlockSpec((NC, BP, D), lambda i: (0, i, 0)),
            pl.BlockSpec((NW, BP), lambda i: (0, i)),
            pl.BlockSpec((D, C), lambda i: (0, 0)),
            pl.BlockSpec((1, C), lambda i: (0, 0)),
        ],
        out_specs=pl.BlockSpec((BP, C), lambda i: (i, 0)),
        out_shape=jax.ShapeDtypeStruct((NP, C), jnp.float32),
    )(part2, degp, Wd, bd.reshape(1, C))

    return out[:N]


# speedup vs baseline: 2.6857x; 1.0279x over previous
import jax
import jax.numpy as jnp
from jax import lax
from jax.experimental import pallas as pl
from jax.experimental.pallas import tpu as pltpu
from jax.experimental.pallas import tpu_sc as plsc

N = 10000
E = 320000
D = 128
C = 7

NC = 2
NS = 16
NW = NC * NS
NP = 10240
EPW = E // NW
K = 125
NCH = EPW // K
IB = 16
NIB = NCH // IB
RPT = NP // NS
RCH = 80
NRCH = RPT // RCH
HL = EPW // 16

_mesh = plsc.VectorSubcoreMesh(
    core_axis_name="c", subcore_axis_name="s", num_cores=NC, num_subcores=NS)


def _sc_deg_body(eif_hbm, zflat_hbm, degp_hbm, dstf_v, hist_v):
    c = lax.axis_index("c")
    s = lax.axis_index("s")
    w = c * NS + s

    pltpu.sync_copy(eif_hbm.at[1, w], dstf_v)
    pltpu.sync_copy(zflat_hbm, hist_v)
    ones16 = jnp.ones((16,), jnp.float32)

    def hstep(i, carry):
        idx = dstf_v[pl.ds(pl.multiple_of(i * 16, 16), 16)]
        plsc.addupdate_scatter(hist_v, [idx], ones16)
        return carry

    lax.fori_loop(0, HL, hstep, 0)
    pltpu.sync_copy(hist_v, degp_hbm.at[w])


_sc_deg = pl.kernel(
    _sc_deg_body,
    out_type=jax.ShapeDtypeStruct((NW, NP), jnp.float32),
    mesh=_mesh,
    scratch_types=[
        pltpu.VMEM((EPW,), jnp.int32),
        pltpu.VMEM((NP,), jnp.float32),
    ],
    compiler_params=pltpu.CompilerParams(needs_layout_passes=False),
)


def _sc_agg_body(h_hbm, ei_hbm, zrow_hbm,
                 part_hbm,
                 src_v, dst_v, rows_v, acc, gsem, ssem):
    c = lax.axis_index("c")
    s = lax.axis_index("s")
    w = c * NS + s

    def start_gather(b, rslot):
        pltpu.async_copy(h_hbm.at[src_v.at[b]], rows_v.at[rslot], gsem)

    def wait_gather(b, rslot):
        pltpu.make_async_copy(h_hbm.at[src_v.at[b]], rows_v.at[rslot],
                              gsem).wait()

    def start_scatter(b, rslot):
        pltpu.async_copy(rows_v.at[rslot], acc.at[dst_v.at[b]], ssem,
                         add=True)

    def wait_scatter(b, rslot):
        pltpu.make_async_copy(rows_v.at[rslot], acc.at[dst_v.at[b]],
                              ssem).wait()

    pltpu.sync_copy(ei_hbm.at[0, w, pl.ds(0, IB)], src_v)
    pltpu.sync_copy(ei_hbm.at[1, w, pl.ds(0, IB)], dst_v)
    start_gather(0, 0)
    pltpu.sync_copy(zrow_hbm, rows_v.at[1, pl.ds(0, RCH)])
    for r in range(NRCH):
        base = s * RPT + r * RCH
        pltpu.sync_copy(rows_v.at[1, pl.ds(0, RCH)],
                        acc.at[pl.ds(base, RCH)])
    plsc.subcore_barrier()

    def group(g, carry):
        gb = pl.multiple_of(g * IB, IB)

        @pl.when(g > 0)
        def _():
            wait_scatter(IB - 1, (IB - 1) % 2)
            pltpu.sync_copy(ei_hbm.at[0, w, pl.ds(gb, IB)], src_v)
            pltpu.sync_copy(ei_hbm.at[1, w, pl.ds(gb, IB)], dst_v)
            start_gather(0, 0)

        for b in range(IB):
            if b > 0:
                wait_scatter(b - 1, (b - 1) % 2)
            wait_gather(b, b % 2)
            if b + 1 < IB:
                start_gather(b + 1, (b + 1) % 2)
            start_scatter(b, b % 2)
        return carry

    lax.fori_loop(0, NIB, group, 0)
    wait_scatter(IB - 1, (IB - 1) % 2)
    plsc.subcore_barrier()

    def rd(r, rslot):
        base = s * RPT + r * RCH
        return pltpu.make_async_copy(acc.at[pl.ds(base, RCH)],
                                     rows_v.at[rslot, pl.ds(0, RCH)], gsem)

    pltpu.sync_copy(acc.at[pl.ds(s * RPT, RCH)], rows_v.at[0, pl.ds(0, RCH)])
    for r in range(NRCH):
        if r + 1 < NRCH:
            pltpu.async_copy(acc.at[pl.ds(s * RPT + (r + 1) * RCH, RCH)],
                             rows_v.at[(r + 1) % 2, pl.ds(0, RCH)], gsem)
        base = s * RPT + r * RCH
        pltpu.sync_copy(rows_v.at[r % 2, pl.ds(0, RCH)],
                        part_hbm.at[c, pl.ds(base, RCH)])
        if r + 1 < NRCH:
            rd(r + 1, (r + 1) % 2).wait()


_sc_agg = pl.kernel(
    _sc_agg_body,
    out_type=jax.ShapeDtypeStruct((NC, NP, D), jnp.float32),
    mesh=_mesh,
    scratch_types=[
        pltpu.VMEM((IB, K), jnp.int32),
        pltpu.VMEM((IB, K), jnp.int32),
        pltpu.VMEM((2, K, D), jnp.float32),
        pltpu.VMEM_SHARED((NP, D), jnp.float32),
        pltpu.SemaphoreType.DMA,
        pltpu.SemaphoreType.DMA,
    ],
    compiler_params=pltpu.CompilerParams(needs_layout_passes=False),
)


BR = 1000
GRID = N // BR
BP = NP // GRID


def _tc_lin1(x_ref, w_ref, b_ref, o_ref):
    o_ref[...] = (
        jnp.dot(x_ref[...], w_ref[...], preferred_element_type=jnp.float32)
        + b_ref[...])


def _tc_lin2(p_ref, dg_ref, w_ref, b_ref, o_ref):
    agg = p_ref[0] + p_ref[1]
    deg = jnp.maximum(jnp.sum(dg_ref[...], axis=0), 1.0)[:, None]
    h = jnp.maximum(agg / deg, 0.0)
    o_ref[...] = (
        jnp.dot(h, w_ref[...], preferred_element_type=jnp.float32)
        + b_ref[...])


def _tc_decode(p_ref, dg_ref, w_ref, b_ref, o_ref):
    agg = p_ref[0] + p_ref[1]
    deg = jnp.maximum(jnp.sum(dg_ref[...], axis=0), 1.0)[:, None]
    h = agg / deg
    logits = (
        jnp.dot(h, w_ref[...], preferred_element_type=jnp.float32)
        + b_ref[...])
    m = jnp.max(logits, axis=-1, keepdims=True)
    ex = jnp.exp(logits - m)
    lse = jnp.log(jnp.sum(ex, axis=-1, keepdims=True))
    o_ref[...] = logits - m - lse


def kernel(x, edge_index, W1, b1, W2, b2, Wd, bd):
    ei3 = edge_index.reshape(2, NW, NCH, K)
    eif = edge_index.reshape(2, NW, EPW)
    zrow = jnp.zeros((RCH, D), jnp.float32)
    zflat = jnp.zeros((NP,), jnp.float32)

    degp = _sc_deg(eif, zflat)

    h1 = pl.pallas_call(
        _tc_lin1,
        grid=(GRID,),
        in_specs=[
            pl.BlockSpec((BR, D), lambda i: (i, 0)),
            pl.BlockSpec((D, D), lambda i: (0, 0)),
            pl.BlockSpec((1, D), lambda i: (0, 0)),
        ],
        out_specs=pl.BlockSpec((BR, D), lambda i: (i, 0)),
        out_shape=jax.ShapeDtypeStruct((N, D), jnp.float32),
    )(x, W1, b1.reshape(1, D))

    part1 = _sc_agg(h1, ei3, zrow)

    h2 = pl.pallas_call(
        _tc_lin2,
        grid=(GRID,),
        in_specs=[
            pl.BlockSpec((NC, BP, D), lambda i: (0, i, 0)),
            pl.BlockSpec((NW, BP), lambda i: (0, i)),
            pl.BlockSpec((D, D), lambda i: (0, 0)),
            pl.BlockSpec((1, D), lambda i: (0, 0)),
        ],
        out_specs=pl.BlockSpec((BP, D), lambda i: (i, 0)),
        out_shape=jax.ShapeDtypeStruct((NP, D), jnp.float32),
    )(part1, degp, W2, b2.reshape(1, D))

    part2 = _sc_agg(h2, ei3, zrow)

    out = pl.pallas_call(
        _tc_decode,
        grid=(GRID,),
        in_specs=[
            pl.BlockSpec((NC, BP, D), lambda i: (0, i, 0)),
            pl.BlockSpec((NW, BP), lambda i: (0, i)),
            pl.BlockSpec((D, C), lambda i: (0, 0)),
            pl.BlockSpec((1, C), lambda i: (0, 0)),
        ],
        out_specs=pl.BlockSpec((BP, C), lambda i: (i, 0)),
        out_shape=jax.ShapeDtypeStruct((NP, C), jnp.float32),
    )(part2, degp, Wd, bd.reshape(1, C))

    return out[:N]


# speedup vs baseline: 2.7737x; 1.0328x over previous
import jax
import jax.numpy as jnp
from jax import lax
from jax.experimental import pallas as pl
from jax.experimental.pallas import tpu as pltpu
from jax.experimental.pallas import tpu_sc as plsc

N = 10000
E = 320000
D = 128
C = 7

NC = 2
NS = 16
NW = NC * NS
NP = 10240
EPW = E // NW
K = 125
NCH = EPW // K
IB = 40
NIB = NCH // IB
RPT = NP // NS
RCH = 80
NRCH = RPT // RCH
HL = EPW // 16

_mesh = plsc.VectorSubcoreMesh(
    core_axis_name="c", subcore_axis_name="s", num_cores=NC, num_subcores=NS)


def _sc_deg_body(eif_hbm, zflat_hbm, degp_hbm, dstf_v, hist_v):
    c = lax.axis_index("c")
    s = lax.axis_index("s")
    w = c * NS + s

    pltpu.sync_copy(eif_hbm.at[1, w], dstf_v)
    pltpu.sync_copy(zflat_hbm, hist_v)
    ones16 = jnp.ones((16,), jnp.float32)

    def hstep(i, carry):
        idx = dstf_v[pl.ds(pl.multiple_of(i * 16, 16), 16)]
        plsc.addupdate_scatter(hist_v, [idx], ones16)
        return carry

    lax.fori_loop(0, HL, hstep, 0)
    pltpu.sync_copy(hist_v, degp_hbm.at[w])


_sc_deg = pl.kernel(
    _sc_deg_body,
    out_type=jax.ShapeDtypeStruct((NW, NP), jnp.float32),
    mesh=_mesh,
    scratch_types=[
        pltpu.VMEM((EPW,), jnp.int32),
        pltpu.VMEM((NP,), jnp.float32),
    ],
    compiler_params=pltpu.CompilerParams(needs_layout_passes=False),
)


def _sc_agg_body(h_hbm, ei_hbm, zrow_hbm,
                 part_hbm,
                 src_v, dst_v, rows_v, acc, gsem, ssem):
    c = lax.axis_index("c")
    s = lax.axis_index("s")
    w = c * NS + s

    def start_gather(b, rslot):
        pltpu.async_copy(h_hbm.at[src_v.at[b]], rows_v.at[rslot], gsem)

    def wait_gather(b, rslot):
        pltpu.make_async_copy(h_hbm.at[src_v.at[b]], rows_v.at[rslot],
                              gsem).wait()

    def start_scatter(b, rslot):
        pltpu.async_copy(rows_v.at[rslot], acc.at[dst_v.at[b]], ssem,
                         add=True)

    def wait_scatter(b, rslot):
        pltpu.make_async_copy(rows_v.at[rslot], acc.at[dst_v.at[b]],
                              ssem).wait()

    pltpu.sync_copy(ei_hbm.at[0, w, pl.ds(0, IB)], src_v)
    pltpu.sync_copy(ei_hbm.at[1, w, pl.ds(0, IB)], dst_v)
    start_gather(0, 0)
    pltpu.sync_copy(zrow_hbm, rows_v.at[1, pl.ds(0, RCH)])
    for r in range(NRCH):
        base = s * RPT + r * RCH
        pltpu.sync_copy(rows_v.at[1, pl.ds(0, RCH)],
                        acc.at[pl.ds(base, RCH)])
    plsc.subcore_barrier()

    def group(g, carry):
        gb = pl.multiple_of(g * IB, IB)

        @pl.when(g > 0)
        def _():
            wait_scatter(IB - 1, (IB - 1) % 2)
            pltpu.sync_copy(ei_hbm.at[0, w, pl.ds(gb, IB)], src_v)
            pltpu.sync_copy(ei_hbm.at[1, w, pl.ds(gb, IB)], dst_v)
            start_gather(0, 0)

        for b in range(IB):
            if b > 0:
                wait_scatter(b - 1, (b - 1) % 2)
            wait_gather(b, b % 2)
            if b + 1 < IB:
                start_gather(b + 1, (b + 1) % 2)
            start_scatter(b, b % 2)
        return carry

    lax.fori_loop(0, NIB, group, 0)
    wait_scatter(IB - 1, (IB - 1) % 2)
    plsc.subcore_barrier()

    def rd(r, rslot):
        base = s * RPT + r * RCH
        return pltpu.make_async_copy(acc.at[pl.ds(base, RCH)],
                                     rows_v.at[rslot, pl.ds(0, RCH)], gsem)

    pltpu.sync_copy(acc.at[pl.ds(s * RPT, RCH)], rows_v.at[0, pl.ds(0, RCH)])
    for r in range(NRCH):
        if r + 1 < NRCH:
            pltpu.async_copy(acc.at[pl.ds(s * RPT + (r + 1) * RCH, RCH)],
                             rows_v.at[(r + 1) % 2, pl.ds(0, RCH)], gsem)
        base = s * RPT + r * RCH
        pltpu.sync_copy(rows_v.at[r % 2, pl.ds(0, RCH)],
                        part_hbm.at[c, pl.ds(base, RCH)])
        if r + 1 < NRCH:
            rd(r + 1, (r + 1) % 2).wait()


_sc_agg = pl.kernel(
    _sc_agg_body,
    out_type=jax.ShapeDtypeStruct((NC, NP, D), jnp.float32),
    mesh=_mesh,
    scratch_types=[
        pltpu.VMEM((IB, K), jnp.int32),
        pltpu.VMEM((IB, K), jnp.int32),
        pltpu.VMEM((2, K, D), jnp.float32),
        pltpu.VMEM_SHARED((NP, D), jnp.float32),
        pltpu.SemaphoreType.DMA,
        pltpu.SemaphoreType.DMA,
    ],
    compiler_params=pltpu.CompilerParams(needs_layout_passes=False),
)


BR = 1000
GRID = N // BR
BP = NP // GRID


def _tc_lin1(x_ref, w_ref, b_ref, o_ref):
    o_ref[...] = (
        jnp.dot(x_ref[...], w_ref[...], preferred_element_type=jnp.float32)
        + b_ref[...])


def _tc_lin2(p_ref, dg_ref, w_ref, b_ref, o_ref):
    agg = p_ref[0] + p_ref[1]
    deg = jnp.maximum(jnp.sum(dg_ref[...], axis=0), 1.0)[:, None]
    h = jnp.maximum(agg / deg, 0.0)
    o_ref[...] = (
        jnp.dot(h, w_ref[...], preferred_element_type=jnp.float32)
        + b_ref[...])


def _tc_decode(p_ref, dg_ref, w_ref, b_ref, o_ref):
    agg = p_ref[0] + p_ref[1]
    deg = jnp.maximum(jnp.sum(dg_ref[...], axis=0), 1.0)[:, None]
    h = agg / deg
    logits = (
        jnp.dot(h, w_ref[...], preferred_element_type=jnp.float32)
        + b_ref[...])
    m = jnp.max(logits, axis=-1, keepdims=True)
    ex = jnp.exp(logits - m)
    lse = jnp.log(jnp.sum(ex, axis=-1, keepdims=True))
    o_ref[...] = logits - m - lse


def kernel(x, edge_index, W1, b1, W2, b2, Wd, bd):
    ei3 = edge_index.reshape(2, NW, NCH, K)
    eif = edge_index.reshape(2, NW, EPW)
    zrow = jnp.zeros((RCH, D), jnp.float32)
    zflat = jnp.zeros((NP,), jnp.float32)

    degp = _sc_deg(eif, zflat)

    h1 = pl.pallas_call(
        _tc_lin1,
        grid=(GRID,),
        in_specs=[
            pl.BlockSpec((BR, D), lambda i: (i, 0)),
            pl.BlockSpec((D, D), lambda i: (0, 0)),
            pl.BlockSpec((1, D), lambda i: (0, 0)),
        ],
        out_specs=pl.BlockSpec((BR, D), lambda i: (i, 0)),
        out_shape=jax.ShapeDtypeStruct((N, D), jnp.float32),
    )(x, W1, b1.reshape(1, D))

    part1 = _sc_agg(h1, ei3, zrow)

    h2 = pl.pallas_call(
        _tc_lin2,
        grid=(GRID,),
        in_specs=[
            pl.BlockSpec((NC, BP, D), lambda i: (0, i, 0)),
            pl.BlockSpec((NW, BP), lambda i: (0, i)),
            pl.BlockSpec((D, D), lambda i: (0, 0)),
            pl.BlockSpec((1, D), lambda i: (0, 0)),
        ],
        out_specs=pl.BlockSpec((BP, D), lambda i: (i, 0)),
        out_shape=jax.ShapeDtypeStruct((NP, D), jnp.float32),
    )(part1, degp, W2, b2.reshape(1, D))

    part2 = _sc_agg(h2, ei3, zrow)

    out = pl.pallas_call(
        _tc_decode,
        grid=(GRID,),
        in_specs=[
            pl.BlockSpec((NC, BP, D), lambda i: (0, i, 0)),
            pl.BlockSpec((NW, BP), lambda i: (0, i)),
            pl.BlockSpec((D, C), lambda i: (0, 0)),
            pl.BlockSpec((1, C), lambda i: (0, 0)),
        ],
        out_specs=pl.BlockSpec((BP, C), lambda i: (i, 0)),
        out_shape=jax.ShapeDtypeStruct((NP, C), jnp.float32),
    )(part2, degp, Wd, bd.reshape(1, C))

    return out[:N]
